# Initial kernel scaffold; baseline (speedup 1.0000x reference)
import jax
import jax.numpy as jnp
from jax.experimental import pallas as pl


def kernel(sequence, table, t_pos_embed, v_pos_embed):
    raise NotImplementedError("write your pallas kernel here")


# speedup vs baseline: 1.7943x; 1.7943x over previous
import functools

import jax
import jax.numpy as jnp
from jax import lax
from jax.experimental import pallas as pl
from jax.experimental.pallas import tpu as pltpu
from jax.experimental.pallas import tpu_sc as plsc

_VOCAB = 100000
_EMB = 64
_VPOS = 32
_H = _EMB + _VPOS
_NUM_IMAGE = 5
_B = 1024
_T2 = 52
_CTX = (_NUM_IMAGE - 1) * _T2 + 1
_TGT = _CTX + _T2
_PAD = 264
_NCHUNK = 3
_CHUNK = _PAD // _NCHUNK
_MASK_ID, _BOS_ID, _EOS_ID = 0, 1, 2


def _sc_body(tok_hbm, table_hbm, bias_hbm, vpat_hbm, ctx_hbm, tgt_hbm,
             idx_v, g_v, o_v, bias_v, vp_v, sem):
    info = plsc.get_sparse_core_info()
    nc = info.num_cores
    nw = nc * info.num_subcores
    bpw = _B // nw
    wid = lax.axis_index("s") * nc + lax.axis_index("c")
    base = wid * bpw

    pltpu.sync_copy(bias_hbm, bias_v)
    pltpu.sync_copy(vpat_hbm, vp_v)

    def init_row(j, carry):
        for cc in range(_VPOS // 16):
            o_v[j, pl.ds(_EMB + cc * 16, 16)] = vp_v[j, pl.ds(cc * 16, 16)]
        return carry

    lax.fori_loop(0, _TGT, init_row, 0)

    def per_batch(i, carry):
        b = base + i
        pltpu.sync_copy(tok_hbm.at[b], idx_v)
        cps = [
            pltpu.async_copy(table_hbm.at[idx_v.at[k]],
                             g_v.at[pl.ds(k * _CHUNK, _CHUNK)], sem)
            for k in range(_NCHUNK)
        ]
        for cp in cps:
            cp.wait()

        def row(j, inner):
            for cc in range(_EMB // 16):
                sl = pl.ds(cc * 16, 16)
                o_v[j, sl] = g_v[j, sl] + bias_v[j, sl]
            return inner

        lax.fori_loop(0, _TGT, row, 0)
        pltpu.sync_copy(o_v.at[pl.ds(0, _TGT)], tgt_hbm.at[b])
        pltpu.sync_copy(o_v.at[pl.ds(0, _CTX)], ctx_hbm.at[b])
        return carry

    lax.fori_loop(0, bpw, per_batch, 0)


@functools.partial(jax.jit, static_argnames=())
def _sc_encode(tok3, table, bias64, vpat):
    mesh = plsc.VectorSubcoreMesh(core_axis_name="c", subcore_axis_name="s")
    fn = pl.kernel(
        _sc_body,
        out_type=(
            jax.ShapeDtypeStruct((_B, _CTX, _H), jnp.float32),
            jax.ShapeDtypeStruct((_B, _TGT, _H), jnp.float32),
        ),
        mesh=mesh,
        scratch_types=[
            pltpu.VMEM((_NCHUNK, _CHUNK), jnp.int32),
            pltpu.VMEM((_PAD, _EMB), jnp.float32),
            pltpu.VMEM((_PAD, _H), jnp.float32),
            pltpu.VMEM((_PAD, _EMB), jnp.float32),
            pltpu.VMEM((_PAD, _VPOS), jnp.float32),
            pltpu.SemaphoreType.DMA,
        ],
        compiler_params=pltpu.CompilerParams(use_tc_tiling_on_sc=False),
    )
    return fn(tok3, table, bias64, vpat)


def kernel(sequence, table, t_pos_embed, v_pos_embed):
    Bn, Vn, Tn = sequence.shape
    t2 = Tn + 2
    seq = sequence.astype(jnp.int32)
    bos = jnp.full((Bn, Vn, 1), _BOS_ID, jnp.int32)
    eos = jnp.full((Bn, Vn, 1), _EOS_ID, jnp.int32)
    pad_seq = jnp.concatenate([bos, seq, eos], axis=-1)
    tok = jnp.concatenate(
        [pad_seq[:, : Vn - 1].reshape(Bn, (Vn - 1) * t2),
         jnp.full((Bn, 1), _MASK_ID, jnp.int32),
         pad_seq[:, Vn - 1]], axis=1)
    tok3 = jnp.pad(tok, ((0, 0), (0, _PAD - _TGT))).reshape(Bn, _NCHUNK, _CHUNK)

    t52 = t_pos_embed[:t2]
    bias64 = jnp.concatenate(
        [jnp.tile(t52, (Vn - 1, 1)),
         jnp.zeros((1, _EMB), jnp.float32),
         t52,
         jnp.zeros((_PAD - _TGT, _EMB), jnp.float32)], axis=0)
    v5 = v_pos_embed[:Vn]
    vpat = jnp.concatenate(
        [jnp.repeat(v5[: Vn - 1], t2, axis=0),
         jnp.tile(v5[Vn - 1: Vn], (_TGT - (Vn - 1) * t2 + (_PAD - _TGT), 1))],
        axis=0)

    ctx_emb, tgt_emb = _sc_encode(tok3, table, bias64, vpat)

    tgt_seq = tok.astype(sequence.dtype)
    ctx_seq = tgt_seq[:, :_CTX]
    return ctx_emb, ctx_seq, tgt_emb, tgt_seq


# speedup vs baseline: 2.1813x; 1.2156x over previous
import jax
import jax.numpy as jnp
from jax import lax
from jax.experimental import pallas as pl
from jax.experimental.pallas import tpu as pltpu
from jax.experimental.pallas import tpu_sc as plsc

_VOCAB = 100000
_EMB = 64
_VPOS = 32
_H = _EMB + _VPOS
_NUM_IMAGE = 5
_B = 1024
_T2 = 52
_CTX = (_NUM_IMAGE - 1) * _T2 + 1
_TGT = _CTX + _T2
_ROWPAD = 264
_LANES = 128
_BCH = _B // _LANES
_UNITS = _ROWPAD * _BCH
_MASK_ID, _BOS_ID, _EOS_ID = 0, 1, 2
_NBUF = 4
_JSPAN = 9
_TCG = 8


def _sc_gather_body(tok_hbm, table_hbm, out_hbm, idx_v, g_bufs, gsem, osem):
    info = plsc.get_sparse_core_info()
    nc = info.num_cores
    nw = nc * info.num_subcores
    upw = _UNITS // nw
    wid = lax.axis_index("s") * nc + lax.axis_index("c")
    u0 = wid * upw
    j0 = lax.div(u0, _BCH)

    pltpu.sync_copy(tok_hbm.at[pl.ds(j0, _JSPAN)], idx_v)

    def gather_unit(u, buf):
        g = u0 + u
        j = lax.div(g, _BCH) - j0
        bc = lax.rem(g, _BCH)
        pltpu.async_copy(table_hbm.at[idx_v.at[j, bc]], buf, gsem)

    def store_unit(u, buf):
        g = u0 + u
        j = lax.div(g, _BCH)
        bc = lax.rem(g, _BCH)
        pltpu.async_copy(buf, out_hbm.at[j, pl.ds(bc * _LANES, _LANES)], osem)

    def drain_gather(buf):
        pltpu.make_async_copy(table_hbm.at[idx_v.at[0, 0]], buf, gsem).wait()

    def drain_store(buf):
        pltpu.make_async_copy(buf, out_hbm.at[0, pl.ds(0, _LANES)],
                              osem).wait()

    def with_buf(fn, sel):
        for k in range(_NBUF):
            @pl.when(sel == k)
            def _():
                fn(g_bufs[k])

    with_buf(lambda b: gather_unit(0, b), 0)

    def per_unit(u, carry):
        k = lax.rem(u, _NBUF)
        with_buf(drain_gather, k)
        with_buf(lambda b: store_unit(u, b), k)

        @pl.when(u + 1 < upw)
        def _():
            k1 = lax.rem(u + 1, _NBUF)

            @pl.when(u + 1 >= _NBUF)
            def _():
                with_buf(drain_store, k1)

            with_buf(lambda b: gather_unit(u + 1, b), k1)

        return carry

    lax.fori_loop(0, upw, per_unit, 0)
    for t in range(_NBUF - 1):
        with_buf(drain_store, lax.rem(upw - 1 - t, _NBUF))


def _sc_gather(tok3, table_p):
    mesh = plsc.VectorSubcoreMesh(core_axis_name="c", subcore_axis_name="s")
    fn = pl.kernel(
        _sc_gather_body,
        out_type=jax.ShapeDtypeStruct((_ROWPAD, _B, _LANES), jnp.float32),
        mesh=mesh,
        scratch_types=[
            pltpu.VMEM((_JSPAN, _BCH, _LANES), jnp.int32),
            [pltpu.VMEM((_LANES, _LANES), jnp.float32) for _ in range(_NBUF)],
            pltpu.SemaphoreType.DMA,
            pltpu.SemaphoreType.DMA,
        ],
        compiler_params=pltpu.CompilerParams(use_tc_tiling_on_sc=True),
    )
    return fn(tok3, table_p)


def _tc_add_body(g_ref, bias_ref, out_ref):
    x = jnp.swapaxes(g_ref[...], 1, 2)
    out_ref[...] = x[:, :_H, :] + bias_ref[...][:, :_H, None]


def _tc_add(gath, bias, rows):
    grid = (pl.cdiv(rows, _TCG),)
    return pl.pallas_call(
        _tc_add_body,
        grid=grid,
        in_specs=[
            pl.BlockSpec((_TCG, _B, _LANES), lambda i: (i, 0, 0)),
            pl.BlockSpec((_TCG, _LANES), lambda i: (i, 0)),
        ],
        out_specs=pl.BlockSpec((_TCG, _H, _B), lambda i: (i, 0, 0)),
        out_shape=jax.ShapeDtypeStruct((rows, _H, _B), jnp.float32),
        compiler_params=pltpu.CompilerParams(
            dimension_semantics=("arbitrary",)),
    )(gath, bias)


def kernel(sequence, table, t_pos_embed, v_pos_embed):
    Bn, Vn, Tn = sequence.shape
    t2 = Tn + 2
    seq_t = jnp.transpose(sequence.astype(jnp.int32), (1, 2, 0))
    bos = jnp.full((Vn, 1, Bn), _BOS_ID, jnp.int32)
    eos = jnp.full((Vn, 1, Bn), _EOS_ID, jnp.int32)
    pad_seq_t = jnp.concatenate([bos, seq_t, eos], axis=1)
    tok_t = jnp.concatenate(
        [pad_seq_t[: Vn - 1].reshape((Vn - 1) * t2, Bn),
         jnp.full((1, Bn), _MASK_ID, jnp.int32),
         pad_seq_t[Vn - 1]], axis=0)
    tok3 = jnp.pad(tok_t, ((0, _ROWPAD - _TGT), (0, 0))).reshape(
        _ROWPAD, _BCH, _LANES)

    table_p = jnp.pad(table, ((0, 0), (0, _LANES - _EMB)))

    t52 = t_pos_embed[:t2]
    bias64 = jnp.concatenate(
        [jnp.tile(t52, (Vn - 1, 1)),
         jnp.zeros((1, _EMB), jnp.float32),
         t52,
         jnp.zeros((_ROWPAD - _TGT, _EMB), jnp.float32)], axis=0)
    v5 = v_pos_embed[:Vn]
    vpat = jnp.concatenate(
        [jnp.repeat(v5[: Vn - 1], t2, axis=0),
         jnp.tile(v5[Vn - 1: Vn], (_TGT - (Vn - 1) * t2 + (_ROWPAD - _TGT), 1))],
        axis=0)
    bias = jnp.concatenate(
        [bias64, vpat, jnp.zeros((_ROWPAD, _LANES - _H), jnp.float32)], axis=1)

    gath = _sc_gather(tok3, table_p)
    tgt3 = _tc_add(gath, bias, _TGT)
    ctx3 = _tc_add(gath, bias, _CTX)
    tgt_emb = jnp.transpose(tgt3, (2, 0, 1))
    ctx_emb = jnp.transpose(ctx3, (2, 0, 1))

    tgt_seq = jnp.transpose(tok_t).astype(sequence.dtype)
    ctx_seq = tgt_seq[:, :_CTX]
    return ctx_emb, ctx_seq, tgt_emb, tgt_seq


# speedup vs baseline: 2.2665x; 1.0391x over previous
import jax
import jax.numpy as jnp
from jax import lax
from jax.experimental import pallas as pl
from jax.experimental.pallas import tpu as pltpu
from jax.experimental.pallas import tpu_sc as plsc

_VOCAB = 100000
_EMB = 64
_VPOS = 32
_H = _EMB + _VPOS
_NUM_IMAGE = 5
_B = 1024
_T2 = 52
_CTX = (_NUM_IMAGE - 1) * _T2 + 1
_TGT = _CTX + _T2
_ROWPAD = 264
_LANES = 128
_BCH = _B // _LANES
_UNITS = _ROWPAD * _BCH
_MASK_ID, _BOS_ID, _EOS_ID = 0, 1, 2
_NBUF = 6
_AHEAD = 3
_JSPAN = 9
_TCG = 8


def _sc_gather_body(tok_hbm, table_hbm, out_hbm, idx_v, g_bufs, gsem, osem):
    info = plsc.get_sparse_core_info()
    nc = info.num_cores
    nw = nc * info.num_subcores
    upw = _UNITS // nw
    wid = lax.axis_index("s") * nc + lax.axis_index("c")
    u0 = wid * upw
    j0 = lax.div(u0, _BCH)

    pltpu.sync_copy(tok_hbm.at[pl.ds(j0, _JSPAN)], idx_v)

    def gather_unit(u, buf):
        g = u0 + u
        j = lax.div(g, _BCH) - j0
        bc = lax.rem(g, _BCH)
        pltpu.async_copy(table_hbm.at[idx_v.at[j, bc]], buf, gsem)

    def store_unit(u, buf):
        g = u0 + u
        j = lax.div(g, _BCH)
        bc = lax.rem(g, _BCH)
        pltpu.async_copy(buf, out_hbm.at[j, pl.ds(bc * _LANES, _LANES)], osem)

    def drain_gather(buf):
        pltpu.make_async_copy(table_hbm.at[idx_v.at[0, 0]], buf, gsem).wait()

    def drain_store(buf):
        pltpu.make_async_copy(buf, out_hbm.at[0, pl.ds(0, _LANES)],
                              osem).wait()

    def with_buf(fn, sel):
        for k in range(_NBUF):
            @pl.when(sel == k)
            def _():
                fn(g_bufs[k])

    for t in range(_AHEAD):
        with_buf(lambda b, t=t: gather_unit(t, b), t)

    def per_unit(u, carry):
        k = lax.rem(u, _NBUF)
        with_buf(drain_gather, k)
        with_buf(lambda b: store_unit(u, b), k)

        v = u + _AHEAD

        @pl.when(v < upw)
        def _():
            kv = lax.rem(v, _NBUF)

            @pl.when(v >= _NBUF)
            def _():
                with_buf(drain_store, kv)

            with_buf(lambda b: gather_unit(v, b), kv)

        return carry

    lax.fori_loop(0, upw, per_unit, 0)
    for t in range(_NBUF):
        with_buf(drain_store, lax.rem(upw - _NBUF + t, _NBUF))


def _sc_gather(tok3, table_p):
    mesh = plsc.VectorSubcoreMesh(core_axis_name="c", subcore_axis_name="s")
    fn = pl.kernel(
        _sc_gather_body,
        out_type=jax.ShapeDtypeStruct((_ROWPAD, _B, _LANES), jnp.float32),
        mesh=mesh,
        scratch_types=[
            pltpu.VMEM((_JSPAN, _BCH, _LANES), jnp.int32),
            [pltpu.VMEM((_LANES, _LANES), jnp.float32) for _ in range(_NBUF)],
            pltpu.SemaphoreType.DMA,
            pltpu.SemaphoreType.DMA,
        ],
        compiler_params=pltpu.CompilerParams(use_tc_tiling_on_sc=True),
    )
    return fn(tok3, table_p)


def _tc_add_body(g_ref, bias_ref, out_ref):
    x = jnp.swapaxes(g_ref[...], 1, 2)
    out_ref[...] = x[:, :_H, :] + bias_ref[...][:, :_H, None]


def _tc_add(gath, bias, rows):
    grid = (pl.cdiv(rows, _TCG),)
    return pl.pallas_call(
        _tc_add_body,
        grid=grid,
        in_specs=[
            pl.BlockSpec((_TCG, _B, _LANES), lambda i: (i, 0, 0)),
            pl.BlockSpec((_TCG, _LANES), lambda i: (i, 0)),
        ],
        out_specs=pl.BlockSpec((_TCG, _H, _B), lambda i: (i, 0, 0)),
        out_shape=jax.ShapeDtypeStruct((rows, _H, _B), jnp.float32),
        compiler_params=pltpu.CompilerParams(
            dimension_semantics=("arbitrary",)),
    )(gath, bias)


def kernel(sequence, table, t_pos_embed, v_pos_embed):
    Bn, Vn, Tn = sequence.shape
    t2 = Tn + 2
    seq_t = jnp.transpose(sequence.astype(jnp.int32), (1, 2, 0))
    bos = jnp.full((Vn, 1, Bn), _BOS_ID, jnp.int32)
    eos = jnp.full((Vn, 1, Bn), _EOS_ID, jnp.int32)
    pad_seq_t = jnp.concatenate([bos, seq_t, eos], axis=1)
    tok_t = jnp.concatenate(
        [pad_seq_t[: Vn - 1].reshape((Vn - 1) * t2, Bn),
         jnp.full((1, Bn), _MASK_ID, jnp.int32),
         pad_seq_t[Vn - 1]], axis=0)
    tok3 = jnp.pad(tok_t, ((0, _ROWPAD - _TGT), (0, 0))).reshape(
        _ROWPAD, _BCH, _LANES)

    table_p = jnp.pad(table, ((0, 0), (0, _LANES - _EMB)))

    t52 = t_pos_embed[:t2]
    bias64 = jnp.concatenate(
        [jnp.tile(t52, (Vn - 1, 1)),
         jnp.zeros((1, _EMB), jnp.float32),
         t52,
         jnp.zeros((_ROWPAD - _TGT, _EMB), jnp.float32)], axis=0)
    v5 = v_pos_embed[:Vn]
    vpat = jnp.concatenate(
        [jnp.repeat(v5[: Vn - 1], t2, axis=0),
         jnp.tile(v5[Vn - 1: Vn], (_TGT - (Vn - 1) * t2 + (_ROWPAD - _TGT), 1))],
        axis=0)
    bias = jnp.concatenate(
        [bias64, vpat, jnp.zeros((_ROWPAD, _LANES - _H), jnp.float32)], axis=1)

    gath = _sc_gather(tok3, table_p)
    tgt3 = _tc_add(gath, bias, _TGT)
    ctx3 = _tc_add(gath, bias, _CTX)
    tgt_emb = jnp.transpose(tgt3, (2, 0, 1))
    ctx_emb = jnp.transpose(ctx3, (2, 0, 1))

    tgt_seq = jnp.transpose(tok_t).astype(sequence.dtype)
    ctx_seq = tgt_seq[:, :_CTX]
    return ctx_emb, ctx_seq, tgt_emb, tgt_seq


# speedup vs baseline: 2.6848x; 1.1846x over previous
import jax
import jax.numpy as jnp
from jax import lax
from jax.experimental import pallas as pl
from jax.experimental.pallas import tpu as pltpu
from jax.experimental.pallas import tpu_sc as plsc

_VOCAB = 100000
_EMB = 64
_VPOS = 32
_H = _EMB + _VPOS
_NUM_IMAGE = 5
_B = 1024
_T2 = 52
_CTX = (_NUM_IMAGE - 1) * _T2 + 1
_TGT = _CTX + _T2
_ROWPAD = 264
_LANES = 128
_MASK_ID, _BOS_ID, _EOS_ID = 0, 1, 2
_HALF = 16
_TCG = 16


def _sc_gather_body(tok_hbm, table_hbm, out_hbm, idx_v, g_bufs, gsem, osem):
    info = plsc.get_sparse_core_info()
    nc = info.num_cores
    nw = nc * info.num_subcores
    bpw = _B // nw
    wid = lax.axis_index("s") * nc + lax.axis_index("c")
    base = wid * bpw

    pltpu.sync_copy(tok_hbm.at[pl.ds(base, _HALF)], idx_v)

    def gather_batch(i, buf):
        h = lax.rem(i, _HALF)
        pltpu.async_copy(table_hbm.at[idx_v.at[h, 0]],
                         buf.at[pl.ds(0, _LANES)], gsem)
        pltpu.async_copy(table_hbm.at[idx_v.at[h, 1]],
                         buf.at[pl.ds(_LANES, _LANES)], gsem)
        pltpu.async_copy(table_hbm.at[idx_v.at[h, 2, pl.ds(0, 8)]],
                         buf.at[pl.ds(2 * _LANES, 8)], gsem)

    def drain_gather(buf):
        for rows in (_LANES, _LANES, 8):
            pltpu.make_async_copy(
                table_hbm.at[idx_v.at[0, 0, pl.ds(0, rows)]],
                buf.at[pl.ds(0, rows)], gsem).wait()

    def drain_out(buf):
        pltpu.make_async_copy(buf, out_hbm.at[:, 0], osem).wait()

    def with_buf(fn, sel):
        for k in range(2):
            @pl.when(sel == k)
            def _():
                fn(g_bufs[k])

    gather_batch(0, g_bufs[0])

    def per_batch(i, carry):
        cur = lax.rem(i, 2)
        nxt = lax.rem(i + 1, 2)

        with_buf(drain_gather, cur)

        @pl.when(i + 1 == _HALF)
        def _():
            pltpu.sync_copy(tok_hbm.at[pl.ds(base + _HALF, _HALF)], idx_v)

        @pl.when(i > 0)
        def _():
            with_buf(drain_out, nxt)

        @pl.when(i + 1 < bpw)
        def _():
            def launch(buf):
                gather_batch(i + 1, buf)
            with_buf(launch, nxt)

        def store(buf):
            pltpu.async_copy(buf, out_hbm.at[:, base + i], osem)
        with_buf(store, cur)
        return carry

    lax.fori_loop(0, bpw, per_batch, 0)
    with_buf(drain_out, lax.rem(bpw - 1, 2))


def _sc_gather(tok_p, table_p):
    mesh = plsc.VectorSubcoreMesh(core_axis_name="c", subcore_axis_name="s")
    fn = pl.kernel(
        _sc_gather_body,
        out_type=jax.ShapeDtypeStruct((_ROWPAD, _B, _LANES), jnp.float32),
        mesh=mesh,
        scratch_types=[
            pltpu.VMEM((_HALF, 8, _LANES), jnp.int32),
            [pltpu.VMEM((_ROWPAD, _LANES), jnp.float32) for _ in range(2)],
            pltpu.SemaphoreType.DMA,
            pltpu.SemaphoreType.DMA,
        ],
        compiler_params=pltpu.CompilerParams(use_tc_tiling_on_sc=True),
    )
    return fn(tok_p, table_p)


def _tc_add_body(g_ref, bias_ref, out_ref):
    x = jnp.swapaxes(g_ref[...], 1, 2)
    out_ref[...] = x[:, :_H, :] + bias_ref[...][:, :_H, None]


def _tc_add(gath, bias, rows):
    grid = (pl.cdiv(rows, _TCG),)
    return pl.pallas_call(
        _tc_add_body,
        grid=grid,
        in_specs=[
            pl.BlockSpec((_TCG, _B, _LANES), lambda i: (i, 0, 0)),
            pl.BlockSpec((_TCG, _LANES), lambda i: (i, 0)),
        ],
        out_specs=pl.BlockSpec((_TCG, _H, _B), lambda i: (i, 0, 0)),
        out_shape=jax.ShapeDtypeStruct((rows, _H, _B), jnp.float32),
        compiler_params=pltpu.CompilerParams(
            dimension_semantics=("arbitrary",)),
    )(gath, bias)


def kernel(sequence, table, t_pos_embed, v_pos_embed):
    Bn, Vn, Tn = sequence.shape
    t2 = Tn + 2
    seq = sequence.astype(jnp.int32)
    bos = jnp.full((Bn, Vn, 1), _BOS_ID, jnp.int32)
    eos = jnp.full((Bn, Vn, 1), _EOS_ID, jnp.int32)
    pad_seq = jnp.concatenate([bos, seq, eos], axis=-1)
    tok = jnp.concatenate(
        [pad_seq[:, : Vn - 1].reshape(Bn, (Vn - 1) * t2),
         jnp.full((Bn, 1), _MASK_ID, jnp.int32),
         pad_seq[:, Vn - 1]], axis=1)
    tok_p = jnp.pad(tok, ((0, 0), (0, 8 * _LANES - _TGT))).reshape(Bn, 8, _LANES)

    table_p = jnp.pad(table, ((0, 0), (0, _LANES - _EMB)))

    t52 = t_pos_embed[:t2]
    bias64 = jnp.concatenate(
        [jnp.tile(t52, (Vn - 1, 1)),
         jnp.zeros((1, _EMB), jnp.float32),
         t52,
         jnp.zeros((_ROWPAD - _TGT, _EMB), jnp.float32)], axis=0)
    v5 = v_pos_embed[:Vn]
    vpat = jnp.concatenate(
        [jnp.repeat(v5[: Vn - 1], t2, axis=0),
         jnp.tile(v5[Vn - 1: Vn], (_TGT - (Vn - 1) * t2 + (_ROWPAD - _TGT), 1))],
        axis=0)
    bias = jnp.concatenate(
        [bias64, vpat, jnp.zeros((_ROWPAD, _LANES - _H), jnp.float32)], axis=1)

    gath = _sc_gather(tok_p, table_p)
    tgt3 = _tc_add(gath, bias, _TGT)
    ctx3 = _tc_add(gath, bias, _CTX)
    tgt_emb = jnp.transpose(tgt3, (2, 0, 1))
    ctx_emb = jnp.transpose(ctx3, (2, 0, 1))

    tgt_seq = tok.astype(sequence.dtype)
    ctx_seq = tgt_seq[:, :_CTX]
    return ctx_emb, ctx_seq, tgt_emb, tgt_seq
